# baseline (speedup 1.0000x reference)
import functools

import jax
import jax.numpy as jnp
from jax import lax
from jax.experimental import pallas as pl
from jax.experimental.pallas import tpu as pltpu
from jax.experimental.pallas import tpu_sc as plsc

B = 16384
F = 32
MD = 128

_INFO = plsc.get_sparse_core_info()
NC, NS = _INFO.num_cores, _INFO.num_subcores
NW = NC * NS
BPW = B // NW
CH = 128
NCHUNK = BPW // CH


def _sc_gather(user, item, eug, eig, eum, eim):
    mesh = plsc.VectorSubcoreMesh(core_axis_name="c", subcore_axis_name="s")

    @functools.partial(
        pl.kernel,
        out_type=[
            jax.ShapeDtypeStruct((B, F), jnp.float32),
            jax.ShapeDtypeStruct((B, F), jnp.float32),
            jax.ShapeDtypeStruct((B, MD), jnp.float32),
            jax.ShapeDtypeStruct((B, MD), jnp.float32),
        ],
        mesh=mesh,
        compiler_params=pltpu.CompilerParams(use_tc_tiling_on_sc=False),
        scratch_types=[
            pltpu.VMEM((BPW,), jnp.int32),
            pltpu.VMEM((BPW,), jnp.int32),
            pltpu.VMEM((CH, F), jnp.float32),
            pltpu.VMEM((CH, F), jnp.float32),
            pltpu.VMEM((CH, MD), jnp.float32),
            pltpu.VMEM((CH, MD), jnp.float32),
            pltpu.SemaphoreType.DMA,
            pltpu.SemaphoreType.DMA,
            pltpu.SemaphoreType.DMA,
            pltpu.SemaphoreType.DMA,
        ],
    )
    def k(user_hbm, item_hbm, eug_hbm, eig_hbm, eum_hbm, eim_hbm,
          gu_out, gi_out, mu_out, mi_out,
          idx_u, idx_i, gu_buf, gi_buf, mu_buf, mi_buf,
          sem0, sem1, sem2, sem3):
        wid = lax.axis_index("s") * NC + lax.axis_index("c")
        base = wid * BPW
        pltpu.sync_copy(user_hbm.at[pl.ds(base, BPW)], idx_u)
        pltpu.sync_copy(item_hbm.at[pl.ds(base, BPW)], idx_i)
        for c in range(NCHUNK):
            iu = idx_u.at[pl.ds(c * CH, CH)]
            ii = idx_i.at[pl.ds(c * CH, CH)]
            c0 = pltpu.async_copy(eug_hbm.at[iu], gu_buf, sem0)
            c1 = pltpu.async_copy(eig_hbm.at[ii], gi_buf, sem1)
            c2 = pltpu.async_copy(eum_hbm.at[iu], mu_buf, sem2)
            c3 = pltpu.async_copy(eim_hbm.at[ii], mi_buf, sem3)
            c0.wait(); c1.wait(); c2.wait(); c3.wait()
            row = base + c * CH
            pltpu.sync_copy(gu_buf, gu_out.at[pl.ds(row, CH)])
            pltpu.sync_copy(gi_buf, gi_out.at[pl.ds(row, CH)])
            pltpu.sync_copy(mu_buf, mu_out.at[pl.ds(row, CH)])
            pltpu.sync_copy(mi_buf, mi_out.at[pl.ds(row, CH)])

    return k(user, item, eug, eig, eum, eim)


_RB = 1024
_NBLK = B // _RB


def _tc_body(gu, gi, mu, mi, w1a, w1b, b1, w2, b2, w3, b3, wpg, wpm, bp, out):
    h = jnp.maximum(
        jnp.dot(mu[...], w1a[...], preferred_element_type=jnp.float32)
        + jnp.dot(mi[...], w1b[...], preferred_element_type=jnp.float32)
        + b1[...], 0.0)
    h = jnp.maximum(
        jnp.dot(h, w2[...], preferred_element_type=jnp.float32) + b2[...], 0.0)
    h = jnp.maximum(
        jnp.dot(h, w3[...], preferred_element_type=jnp.float32) + b3[...], 0.0)
    g = gu[...] * gi[...]
    pred = jnp.sum(g * wpg[...], axis=1) + jnp.sum(h * wpm[...], axis=1) + bp[0, 0]
    out[...] = pred.reshape(_RB, 1)


def _tc_dense(gu, gi, mu, mi, W1, b1, W2, b2, W3, b3, Wp, bp):
    w1a = W1[:MD]
    w1b = W1[MD:]
    wpg = Wp[:F].reshape(1, F)
    wpm = Wp[F:].reshape(1, F)
    full = lambda shape: pl.BlockSpec(shape, lambda i: (0, 0))
    out = pl.pallas_call(
        _tc_body,
        grid=(_NBLK,),
        in_specs=[
            pl.BlockSpec((_RB, F), lambda i: (i, 0)),
            pl.BlockSpec((_RB, F), lambda i: (i, 0)),
            pl.BlockSpec((_RB, MD), lambda i: (i, 0)),
            pl.BlockSpec((_RB, MD), lambda i: (i, 0)),
            full((MD, MD)),
            full((MD, MD)),
            full((1, MD)),
            full((MD, 64)),
            full((1, 64)),
            full((64, F)),
            full((1, F)),
            full((1, F)),
            full((1, F)),
            full((1, 1)),
        ],
        out_specs=pl.BlockSpec((_RB, 1), lambda i: (i, 0)),
        out_shape=jax.ShapeDtypeStruct((B, 1), jnp.float32),
    )(gu, gi, mu, mi, w1a, w1b, b1.reshape(1, MD), W2, b2.reshape(1, 64),
      W3, b3.reshape(1, F), wpg, wpm, bp.reshape(1, 1))
    return out.reshape(-1)


def kernel(user, item, embed_user_GMF, embed_item_GMF, embed_user_MLP,
           embed_item_MLP, W1, b1, W2, b2, W3, b3, Wp, bp):
    user = user.astype(jnp.int32)
    item = item.astype(jnp.int32)
    gu, gi, mu, mi = _sc_gather(user, item, embed_user_GMF, embed_item_GMF,
                                embed_user_MLP, embed_item_MLP)
    return _tc_dense(gu, gi, mu, mi, W1, b1, W2, b2, W3, b3, Wp, bp)


# speedup vs baseline: 1.4616x; 1.4616x over previous
import functools

import jax
import jax.numpy as jnp
from jax import lax
from jax.experimental import pallas as pl
from jax.experimental.pallas import tpu as pltpu
from jax.experimental.pallas import tpu_sc as plsc

B = 16384
F = 32
MD = 128

_INFO = plsc.get_sparse_core_info()
NC, NS, L = _INFO.num_cores, _INFO.num_subcores, _INFO.num_lanes
NW = NC * NS
BPW = B // NW
CH = 128
NCHUNK = BPW // CH
CHG = 256
NBATCH = BPW // CHG


def _sc_gather(user, item, eug, eig, eum, eim):
    mesh = plsc.VectorSubcoreMesh(core_axis_name="c", subcore_axis_name="s")

    @functools.partial(
        pl.kernel,
        out_type=[
            jax.ShapeDtypeStruct((B, F), jnp.float32),
            jax.ShapeDtypeStruct((B, F), jnp.float32),
            jax.ShapeDtypeStruct((B, MD), jnp.float32),
            jax.ShapeDtypeStruct((B, MD), jnp.float32),
        ],
        mesh=mesh,
        compiler_params=pltpu.CompilerParams(needs_layout_passes=False),
        scratch_types=[
            pltpu.VMEM((BPW,), jnp.int32),
            pltpu.VMEM((BPW,), jnp.int32),
            pltpu.VMEM((CH, MD), jnp.float32),
            pltpu.VMEM((CH, MD), jnp.float32),
            pltpu.VMEM((CHG, F), jnp.float32),
            pltpu.VMEM((CHG, F), jnp.float32),
            pltpu.SemaphoreType.DMA,
            pltpu.SemaphoreType.DMA,
            pltpu.SemaphoreType.DMA,
            pltpu.SemaphoreType.DMA,
        ],
    )
    def k(user_hbm, item_hbm, eug_hbm, eig_hbm, eum_hbm, eim_hbm,
          gu_out, gi_out, mu_out, mi_out,
          idx_u, idx_i, mu_buf, mi_buf, gu_rows, gi_rows,
          sem0, sem1, sem2, sem3):
        wid = lax.axis_index("s") * NC + lax.axis_index("c")
        base = wid * BPW
        pltpu.sync_copy(user_hbm.at[pl.ds(base, BPW)], idx_u)
        pltpu.sync_copy(item_hbm.at[pl.ds(base, BPW)], idx_i)
        for c in range(NCHUNK):
            iu = idx_u.at[pl.ds(c * CH, CH)]
            ii = idx_i.at[pl.ds(c * CH, CH)]
            c0 = pltpu.async_copy(eum_hbm.at[iu], mu_buf, sem0)
            c1 = pltpu.async_copy(eim_hbm.at[ii], mi_buf, sem1)
            c0.wait(); c1.wait()
            row = base + c * CH
            pltpu.sync_copy(mu_buf, mu_out.at[pl.ds(row, CH)])
            pltpu.sync_copy(mi_buf, mi_out.at[pl.ds(row, CH)])
        lane = lax.iota(jnp.int32, L)
        zero = jnp.zeros((L,), jnp.int32)
        for hb in range(NBATCH):
            off = hb * CHG

            @pl.loop(0, CHG)
            def _issue(j):
                jm = lax.rem(j, L)
                jb = j - jm
                uvec = idx_u[pl.ds(off + jb, L)]
                ivec = idx_i[pl.ds(off + jb, L)]
                sel = lane == jnp.full((L,), jm, jnp.int32)
                u = lax.reduce_sum(jnp.where(sel, uvec, zero), axes=(0,))
                iv = lax.reduce_sum(jnp.where(sel, ivec, zero), axes=(0,))
                pltpu.async_copy(eug_hbm.at[pl.ds(u, 1)],
                                 gu_rows.at[pl.ds(j, 1)], sem2)
                pltpu.async_copy(eig_hbm.at[pl.ds(iv, 1)],
                                 gi_rows.at[pl.ds(j, 1)], sem3)

            @pl.loop(0, CHG)
            def _drain(j):
                pltpu.make_async_copy(eug_hbm.at[pl.ds(0, 1)],
                                      gu_rows.at[pl.ds(j, 1)], sem2).wait()
                pltpu.make_async_copy(eig_hbm.at[pl.ds(0, 1)],
                                      gi_rows.at[pl.ds(j, 1)], sem3).wait()

            pltpu.sync_copy(gu_rows, gu_out.at[pl.ds(base + off, CHG)])
            pltpu.sync_copy(gi_rows, gi_out.at[pl.ds(base + off, CHG)])

    return k(user, item, eug, eig, eum, eim)


_RB = 1024
_NBLK = B // _RB


def _tc_body(gu, gi, mu, mi, w1a, w1b, b1, w2, b2, w3, b3, wpg, wpm, bp, out):
    h = jnp.maximum(
        jnp.dot(mu[...], w1a[...], preferred_element_type=jnp.float32)
        + jnp.dot(mi[...], w1b[...], preferred_element_type=jnp.float32)
        + b1[...], 0.0)
    h = jnp.maximum(
        jnp.dot(h, w2[...], preferred_element_type=jnp.float32) + b2[...], 0.0)
    h = jnp.maximum(
        jnp.dot(h, w3[...], preferred_element_type=jnp.float32) + b3[...], 0.0)
    g = gu[...] * gi[...]
    pred = (jnp.sum(g * wpg[...], axis=1)
            + jnp.sum(h * wpm[...], axis=1) + bp[0, 0])
    out[...] = pred.reshape(_RB, 1)


def _tc_dense(gu, gi, mu, mi, W1, b1, W2, b2, W3, b3, Wp, bp):
    w1a = W1[:MD]
    w1b = W1[MD:]
    wpg = Wp[:F].reshape(1, F)
    wpm = Wp[F:].reshape(1, F)
    full = lambda shape: pl.BlockSpec(shape, lambda i: (0, 0))
    out = pl.pallas_call(
        _tc_body,
        grid=(_NBLK,),
        in_specs=[
            pl.BlockSpec((_RB, F), lambda i: (i, 0)),
            pl.BlockSpec((_RB, F), lambda i: (i, 0)),
            pl.BlockSpec((_RB, MD), lambda i: (i, 0)),
            pl.BlockSpec((_RB, MD), lambda i: (i, 0)),
            full((MD, MD)),
            full((MD, MD)),
            full((1, MD)),
            full((MD, 64)),
            full((1, 64)),
            full((64, F)),
            full((1, F)),
            full((1, F)),
            full((1, F)),
            full((1, 1)),
        ],
        out_specs=pl.BlockSpec((_RB, 1), lambda i: (i, 0)),
        out_shape=jax.ShapeDtypeStruct((B, 1), jnp.float32),
    )(gu, gi, mu, mi, w1a, w1b, b1.reshape(1, MD), W2, b2.reshape(1, 64),
      W3, b3.reshape(1, F), wpg, wpm, bp.reshape(1, 1))
    return out.reshape(-1)


def kernel(user, item, embed_user_GMF, embed_item_GMF, embed_user_MLP,
           embed_item_MLP, W1, b1, W2, b2, W3, b3, Wp, bp):
    user = user.astype(jnp.int32)
    item = item.astype(jnp.int32)
    gu, gi, mu, mi = _sc_gather(user, item, embed_user_GMF, embed_item_GMF,
                                embed_user_MLP, embed_item_MLP)
    return _tc_dense(gu, gi, mu, mi, W1, b1, W2, b2, W3, b3, Wp, bp)
